# Initial kernel scaffold; baseline (speedup 1.0000x reference)
import jax
import jax.numpy as jnp
from jax.experimental import pallas as pl


def kernel(pos, edge_index, freqs):
    raise NotImplementedError("write your pallas kernel here")


# speedup vs baseline: 6.8887x; 6.8887x over previous
import functools
import math

import jax
import jax.numpy as jnp
from jax import lax
from jax.experimental import pallas as pl
from jax.experimental.pallas import tpu as pltpu
from jax.experimental.pallas import tpu_sc as plsc

CUTOFF = 5.0

_NC = 2
_NS = 16
_NW = _NC * _NS

_K = 40
_LANES = 128


def _sc_d2_body(p4_hbm, src_hbm, dst_hbm, out_hbm,
                sidx, didx, arows, brows, d2v, sem, *, nchunk):
    wid = lax.axis_index("s") * _NC + lax.axis_index("c")
    nt = (nchunk - 1 - wid) // _NW + 1

    def chunk_body(t, carry):
        c = wid + t * _NW
        base = c * _K
        pltpu.sync_copy(src_hbm.at[pl.ds(base, _K)], sidx)
        pltpu.sync_copy(dst_hbm.at[pl.ds(base, _K)], didx)

        def gather5(t5, carry2):
            cps = []
            for u in range(5):
                j = t5 * 5 + u
                cps.append(pltpu.make_async_copy(
                    p4_hbm.at[sidx.at[j]], arows.at[j], sem))
                cps.append(pltpu.make_async_copy(
                    p4_hbm.at[didx.at[j]], brows.at[j], sem))
            for cp in cps:
                cp.start()
            for cp in cps:
                cp.wait()
            return carry2

        lax.fori_loop(0, _K // 5, gather5, 0)

        def compute(q, carry2):
            j = q // 8
            l0 = (q % 8) * 16
            rows = l0 + lax.iota(jnp.int32, 16)
            jf = jnp.full((16,), 0, jnp.int32) + j
            acc = jnp.zeros((16,), jnp.float32)
            for comp_i in range(3):
                cf = jnp.full((16,), comp_i, jnp.int32)
                av = plsc.load_gather(arows, [jf, rows, cf])
                bv = plsc.load_gather(brows, [jf, rows, cf])
                dv = av - bv
                acc = acc + dv * dv
            d2v[j, pl.ds(l0, 16)] = acc
            return carry2

        lax.fori_loop(0, _K * 8, compute, 0)
        pltpu.sync_copy(d2v, out_hbm.at[pl.ds(base, _K)])
        return carry

    lax.fori_loop(0, nt, chunk_body, 0)


def _sc_d2(p4, src2d, dst2d):
    rows = src2d.shape[0]
    nchunk = rows // _K
    mesh = plsc.VectorSubcoreMesh(
        core_axis_name="c", subcore_axis_name="s",
        num_cores=_NC, num_subcores=_NS)
    fn = pl.kernel(
        functools.partial(_sc_d2_body, nchunk=nchunk),
        out_type=jax.ShapeDtypeStruct((rows, _LANES), jnp.float32),
        mesh=mesh,
        compiler_params=pltpu.CompilerParams(
            needs_layout_passes=False, use_tc_tiling_on_sc=False),
        scratch_types=[
            pltpu.VMEM((_K, _LANES), jnp.int32),
            pltpu.VMEM((_K, _LANES), jnp.int32),
            pltpu.VMEM((_K, _LANES, 4), jnp.float32),
            pltpu.VMEM((_K, _LANES, 4), jnp.float32),
            pltpu.VMEM((_K, _LANES), jnp.float32),
            pltpu.SemaphoreType.DMA,
        ],
    )
    return fn(p4, src2d, dst2d)


def _sin_poly(x):
    pi_hi = jnp.float32(3.1415927)
    pi_lo = jnp.float32(-8.742278e-8)
    k = jnp.floor(x * jnp.float32(1.0 / math.pi) + jnp.float32(0.5))
    r = x - k * pi_hi
    r = r - k * pi_lo
    sign = jnp.float32(1.0) - jnp.float32(2.0) * (
        k.astype(jnp.int32) & 1).astype(jnp.float32)
    r2 = r * r
    p = jnp.float32(2.7557314e-6)
    p = p * r2 + jnp.float32(-1.9841270e-4)
    p = p * r2 + jnp.float32(8.3333333e-3)
    p = p * r2 + jnp.float32(-1.6666667e-1)
    p = p * r2 + jnp.float32(1.0)
    return sign * r * p


def _rbf_body(d2_ref, pf_ref, p_ref, out_ref, *, norm):
    d2 = d2_ref[...]
    d = jnp.sqrt(d2)
    invd = lax.rsqrt(d2) * jnp.float32(norm)
    dn = (((1,), (0,)), ((), ()))
    arg = lax.dot_general(d, pf_ref[...], dn, preferred_element_type=jnp.float32)
    invdrep = lax.dot_general(invd, p_ref[...], dn,
                              preferred_element_type=jnp.float32)
    out_ref[...] = _sin_poly(arg) * invdrep


def _rbf(d2_rs, pfmat, pmat, norm, rb):
    e2, g = d2_rs.shape
    w = pmat.shape[1]
    return pl.pallas_call(
        functools.partial(_rbf_body, norm=norm),
        grid=(e2 // rb,),
        in_specs=[
            pl.BlockSpec((rb, g), lambda i: (i, 0)),
            pl.BlockSpec((g, w), lambda i: (0, 0)),
            pl.BlockSpec((g, w), lambda i: (0, 0)),
        ],
        out_specs=pl.BlockSpec((rb, w), lambda i: (i, 0)),
        out_shape=jax.ShapeDtypeStruct((e2, w), jnp.float32),
    )(d2_rs, pfmat, pmat)


def kernel(pos, edge_index, freqs):
    n = pos.shape[0]
    e = edge_index.shape[1]
    nb = freqs.shape[0]

    p4 = jnp.pad(pos, ((0, 0), (0, 1)))
    src2d = edge_index[0].reshape(e // _LANES, _LANES)
    dst2d = edge_index[1].reshape(e // _LANES, _LANES)

    d2 = _sc_d2(p4, src2d, dst2d)

    g = 32
    w = g * nb
    e2 = e // g
    d2_rs = d2.reshape(e2, g)

    pmat = (jnp.arange(w, dtype=jnp.int32)[None, :] // nb
            == jnp.arange(g, dtype=jnp.int32)[:, None]).astype(jnp.float32)
    fp = jnp.tile(freqs, g) * (1.0 / CUTOFF)
    pfmat = pmat * fp[None, :]
    norm = math.sqrt(2.0 / CUTOFF)

    rb = 800
    while e2 % rb:
        rb //= 2
    out2 = _rbf(d2_rs, pfmat, pmat, norm, rb)
    return out2.reshape(e, nb)


# speedup vs baseline: 6.9225x; 1.0049x over previous
import functools
import math

import jax
import jax.numpy as jnp
from jax import lax
from jax.experimental import pallas as pl
from jax.experimental.pallas import tpu as pltpu
from jax.experimental.pallas import tpu_sc as plsc

CUTOFF = 5.0

_NC = 2
_NS = 16
_NW = _NC * _NS

_K = 40
_LANES = 128
_CB = _K * _LANES


def _sc_d2_body(p4_hbm, src_hbm, dst_hbm, out_hbm,
                sidx, didx, arows, brows, d2v, sem, *, nchunk):
    wid = lax.axis_index("s") * _NC + lax.axis_index("c")
    nt = (nchunk - 1 - wid) // _NW + 1

    def chunk_body(t, carry):
        c = wid + t * _NW
        base = c * _CB
        pltpu.sync_copy(src_hbm.at[pl.ds(base, _CB)], sidx)
        pltpu.sync_copy(dst_hbm.at[pl.ds(base, _CB)], didx)

        def gather5(t5, carry2):
            cps = []
            for u in range(5):
                j = t5 * 5 + u
                cps.append(pltpu.make_async_copy(
                    p4_hbm.at[sidx.at[pl.ds(j * _LANES, _LANES)]],
                    arows.at[j], sem))
                cps.append(pltpu.make_async_copy(
                    p4_hbm.at[didx.at[pl.ds(j * _LANES, _LANES)]],
                    brows.at[j], sem))
            for cp in cps:
                cp.start()
            for cp in cps:
                cp.wait()
            return carry2

        lax.fori_loop(0, _K // 5, gather5, 0)

        def compute(q, carry2):
            j = q // 8
            l0 = (q % 8) * 16
            rows = l0 + lax.iota(jnp.int32, 16)
            jf = jnp.full((16,), 0, jnp.int32) + j
            acc = jnp.zeros((16,), jnp.float32)
            for comp_i in range(3):
                cf = jnp.full((16,), comp_i, jnp.int32)
                av = plsc.load_gather(arows, [jf, rows, cf])
                bv = plsc.load_gather(brows, [jf, rows, cf])
                dv = av - bv
                acc = acc + dv * dv
            d2v[pl.ds(q * 16, 16)] = acc
            return carry2

        lax.fori_loop(0, _K * 8, compute, 0)
        pltpu.sync_copy(d2v, out_hbm.at[pl.ds(base, _CB)])
        return carry

    lax.fori_loop(0, nt, chunk_body, 0)


def _sc_d2(p4, src, dst):
    e = src.shape[0]
    nchunk = e // _CB
    mesh = plsc.VectorSubcoreMesh(
        core_axis_name="c", subcore_axis_name="s",
        num_cores=_NC, num_subcores=_NS)
    fn = pl.kernel(
        functools.partial(_sc_d2_body, nchunk=nchunk),
        out_type=jax.ShapeDtypeStruct((e,), jnp.float32),
        mesh=mesh,
        compiler_params=pltpu.CompilerParams(
            needs_layout_passes=False, use_tc_tiling_on_sc=False),
        scratch_types=[
            pltpu.VMEM((_CB,), jnp.int32),
            pltpu.VMEM((_CB,), jnp.int32),
            pltpu.VMEM((_K, _LANES, 4), jnp.float32),
            pltpu.VMEM((_K, _LANES, 4), jnp.float32),
            pltpu.VMEM((_CB,), jnp.float32),
            pltpu.SemaphoreType.DMA,
        ],
    )
    return fn(p4, src, dst)


def _sin_poly(x):
    pi_hi = jnp.float32(3.1415927)
    pi_lo = jnp.float32(-8.742278e-8)
    k = jnp.floor(x * jnp.float32(1.0 / math.pi) + jnp.float32(0.5))
    r = x - k * pi_hi
    r = r - k * pi_lo
    sign = jnp.float32(1.0) - jnp.float32(2.0) * (
        k.astype(jnp.int32) & 1).astype(jnp.float32)
    r2 = r * r
    p = jnp.float32(2.7557314e-6)
    p = p * r2 + jnp.float32(-1.9841270e-4)
    p = p * r2 + jnp.float32(8.3333333e-3)
    p = p * r2 + jnp.float32(-1.6666667e-1)
    p = p * r2 + jnp.float32(1.0)
    return sign * r * p


def _rbf_body(d2_ref, pf_ref, p_ref, out_ref, *, norm, rb):
    d2 = d2_ref[...].reshape(rb, _LANES)
    d = jnp.sqrt(d2)
    invd = lax.rsqrt(d2) * jnp.float32(norm)
    dn = (((1,), (0,)), ((), ()))
    arg = lax.dot_general(d, pf_ref[...], dn, preferred_element_type=jnp.float32)
    invdrep = lax.dot_general(invd, p_ref[...], dn,
                              preferred_element_type=jnp.float32)
    out_ref[...] = _sin_poly(arg) * invdrep


def _rbf(d2, pfmat, pmat, norm, rb):
    e = d2.shape[0]
    e2 = e // _LANES
    w = pmat.shape[1]
    return pl.pallas_call(
        functools.partial(_rbf_body, norm=norm, rb=rb),
        grid=(e2 // rb,),
        in_specs=[
            pl.BlockSpec((rb * _LANES,), lambda i: (i,)),
            pl.BlockSpec((_LANES, w), lambda i: (0, 0)),
            pl.BlockSpec((_LANES, w), lambda i: (0, 0)),
        ],
        out_specs=pl.BlockSpec((rb, w), lambda i: (i, 0)),
        out_shape=jax.ShapeDtypeStruct((e2, w), jnp.float32),
    )(d2, pfmat, pmat)


def kernel(pos, edge_index, freqs):
    e = edge_index.shape[1]
    nb = freqs.shape[0]

    p4 = jnp.pad(pos, ((0, 0), (0, 1)))
    src = edge_index[0]
    dst = edge_index[1]

    d2 = _sc_d2(p4, src, dst)

    g = _LANES
    w = g * nb

    pmat = (jnp.arange(w, dtype=jnp.int32)[None, :] // nb
            == jnp.arange(g, dtype=jnp.int32)[:, None]).astype(jnp.float32)
    fp = jnp.tile(freqs, g) * (1.0 / CUTOFF)
    pfmat = pmat * fp[None, :]
    norm = math.sqrt(2.0 / CUTOFF)

    e2 = e // g
    rb = 200
    while e2 % rb:
        rb //= 2
    out2 = _rbf(d2, pfmat, pmat, norm, rb)
    return out2.reshape(e, nb)


# speedup vs baseline: 6.9314x; 1.0013x over previous
import functools
import math

import jax
import jax.numpy as jnp
from jax import lax
from jax.experimental import pallas as pl
from jax.experimental.pallas import tpu as pltpu
from jax.experimental.pallas import tpu_sc as plsc

CUTOFF = 5.0

_NC = 2
_NS = 16
_NW = _NC * _NS

_K = 40
_LANES = 128
_CB = _K * _LANES


def _sc_d2_body(p4_hbm, src_hbm, dst_hbm, out_hbm,
                sidx, didx, arows, brows, d2v, sem, *, nchunk):
    wid = lax.axis_index("s") * _NC + lax.axis_index("c")
    nt = (nchunk - 1 - wid) // _NW + 1

    def chunk_body(t, carry):
        c = wid + t * _NW
        base = c * _CB
        pltpu.sync_copy(src_hbm.at[pl.ds(base, _CB)], sidx)
        pltpu.sync_copy(dst_hbm.at[pl.ds(base, _CB)], didx)

        def gather5(t5, carry2):
            cps = []
            for u in range(5):
                j = t5 * 5 + u
                cps.append(pltpu.make_async_copy(
                    p4_hbm.at[sidx.at[pl.ds(j * _LANES, _LANES)]],
                    arows.at[j], sem))
                cps.append(pltpu.make_async_copy(
                    p4_hbm.at[didx.at[pl.ds(j * _LANES, _LANES)]],
                    brows.at[j], sem))
            for cp in cps:
                cp.start()
            for cp in cps:
                cp.wait()
            return carry2

        lax.fori_loop(0, _K // 5, gather5, 0)

        def compute(q, carry2):
            j = q // 8
            l0 = (q % 8) * 16
            rows = l0 + lax.iota(jnp.int32, 16)
            jf = jnp.full((16,), 0, jnp.int32) + j
            acc = jnp.zeros((16,), jnp.float32)
            for comp_i in range(3):
                cf = jnp.full((16,), comp_i, jnp.int32)
                av = plsc.load_gather(arows, [jf, rows, cf])
                bv = plsc.load_gather(brows, [jf, rows, cf])
                dv = av - bv
                acc = acc + dv * dv
            d2v[pl.ds(q * 16, 16)] = acc
            return carry2

        lax.fori_loop(0, _K * 8, compute, 0)
        pltpu.sync_copy(d2v, out_hbm.at[pl.ds(base, _CB)])
        return carry

    lax.fori_loop(0, nt, chunk_body, 0)


def _sc_d2(p4, src, dst):
    e = src.shape[0]
    nchunk = e // _CB
    mesh = plsc.VectorSubcoreMesh(
        core_axis_name="c", subcore_axis_name="s",
        num_cores=_NC, num_subcores=_NS)
    fn = pl.kernel(
        functools.partial(_sc_d2_body, nchunk=nchunk),
        out_type=jax.ShapeDtypeStruct((e,), jnp.float32),
        mesh=mesh,
        compiler_params=pltpu.CompilerParams(
            needs_layout_passes=False, use_tc_tiling_on_sc=False),
        scratch_types=[
            pltpu.VMEM((_CB,), jnp.int32),
            pltpu.VMEM((_CB,), jnp.int32),
            pltpu.VMEM((_K, _LANES, 4), jnp.float32),
            pltpu.VMEM((_K, _LANES, 4), jnp.float32),
            pltpu.VMEM((_CB,), jnp.float32),
            pltpu.SemaphoreType.DMA,
        ],
    )
    return fn(p4, src, dst)


def _split_body(ei_ref, src_ref, dst_ref):
    src_ref[...] = ei_ref[0, :]
    dst_ref[...] = ei_ref[1, :]


def _split_edges(edge_index, sb):
    e = edge_index.shape[1]
    out = jax.ShapeDtypeStruct((e,), jnp.int32)
    return pl.pallas_call(
        _split_body,
        grid=(e // sb,),
        in_specs=[pl.BlockSpec((2, sb), lambda i: (0, i))],
        out_specs=[pl.BlockSpec((sb,), lambda i: (i,)),
                   pl.BlockSpec((sb,), lambda i: (i,))],
        out_shape=[out, out],
    )(edge_index)


def _sin_poly(x):
    pi_hi = jnp.float32(3.1415927)
    pi_lo = jnp.float32(-8.742278e-8)
    k = jnp.floor(x * jnp.float32(1.0 / math.pi) + jnp.float32(0.5))
    r = x - k * pi_hi
    r = r - k * pi_lo
    sign = jnp.float32(1.0) - jnp.float32(2.0) * (
        k.astype(jnp.int32) & 1).astype(jnp.float32)
    r2 = r * r
    p = jnp.float32(2.7557314e-6)
    p = p * r2 + jnp.float32(-1.9841270e-4)
    p = p * r2 + jnp.float32(8.3333333e-3)
    p = p * r2 + jnp.float32(-1.6666667e-1)
    p = p * r2 + jnp.float32(1.0)
    return sign * r * p


def _rbf_body(d2_ref, pf_ref, p_ref, out_ref, *, norm, rb):
    d2 = d2_ref[...].reshape(rb, _LANES)
    d = jnp.sqrt(d2)
    invd = lax.rsqrt(d2) * jnp.float32(norm)
    dn = (((1,), (0,)), ((), ()))
    arg = lax.dot_general(d, pf_ref[...], dn, preferred_element_type=jnp.float32)
    invdrep = lax.dot_general(invd, p_ref[...], dn,
                              preferred_element_type=jnp.float32)
    out_ref[...] = _sin_poly(arg) * invdrep


def _rbf(d2, pfmat, pmat, norm, rb):
    e = d2.shape[0]
    e2 = e // _LANES
    w = pmat.shape[1]
    return pl.pallas_call(
        functools.partial(_rbf_body, norm=norm, rb=rb),
        grid=(e2 // rb,),
        in_specs=[
            pl.BlockSpec((rb * _LANES,), lambda i: (i,)),
            pl.BlockSpec((_LANES, w), lambda i: (0, 0)),
            pl.BlockSpec((_LANES, w), lambda i: (0, 0)),
        ],
        out_specs=pl.BlockSpec((rb, w), lambda i: (i, 0)),
        out_shape=jax.ShapeDtypeStruct((e2, w), jnp.float32),
    )(d2, pfmat, pmat)


def kernel(pos, edge_index, freqs):
    e = edge_index.shape[1]
    nb = freqs.shape[0]

    p4 = jnp.pad(pos, ((0, 0), (0, 1)))
    src, dst = _split_edges(edge_index, 128000)

    d2 = _sc_d2(p4, src, dst)

    g = _LANES
    w = g * nb

    pmat = (jnp.arange(w, dtype=jnp.int32)[None, :] // nb
            == jnp.arange(g, dtype=jnp.int32)[:, None]).astype(jnp.float32)
    fp = jnp.tile(freqs, g) * (1.0 / CUTOFF)
    pfmat = pmat * fp[None, :]
    norm = math.sqrt(2.0 / CUTOFF)

    e2 = e // g
    rb = 200
    while e2 % rb:
        rb //= 2
    out2 = _rbf(d2, pfmat, pmat, norm, rb)
    return out2.reshape(e, nb)


# speedup vs baseline: 30.6692x; 4.4247x over previous
import functools
import math

import jax
import jax.numpy as jnp
from jax import lax
from jax.experimental import pallas as pl
from jax.experimental.pallas import tpu as pltpu
from jax.experimental.pallas import tpu_sc as plsc

CUTOFF = 5.0

_NC = 2
_NS = 16
_NW = _NC * _NS

_K = 40
_LANES = 128
_CB = _K * _LANES


def _sc_d2_body(p4_hbm, src_hbm, dst_hbm, out_hbm,
                sidx, didx, arows, brows, d2v, sem, *, nchunk):
    wid = lax.axis_index("s") * _NC + lax.axis_index("c")
    nt = (nchunk - 1 - wid) // _NW + 1

    def chunk_body(t, carry):
        c = wid + t * _NW
        base = c * _CB
        pltpu.sync_copy(src_hbm.at[pl.ds(base, _CB)], sidx)
        pltpu.sync_copy(dst_hbm.at[pl.ds(base, _CB)], didx)

        def gather5(t5, carry2):
            cps = []
            for u in range(5):
                j = t5 * 5 + u
                cps.append(pltpu.make_async_copy(
                    p4_hbm.at[sidx.at[pl.ds(j * _LANES, _LANES)]],
                    arows.at[j], sem))
                cps.append(pltpu.make_async_copy(
                    p4_hbm.at[didx.at[pl.ds(j * _LANES, _LANES)]],
                    brows.at[j], sem))
            for cp in cps:
                cp.start()
            for cp in cps:
                cp.wait()
            return carry2

        lax.fori_loop(0, _K // 5, gather5, 0)

        def compute(q, carry2):
            j = q // 8
            l0 = (q % 8) * 16
            rows = l0 + lax.iota(jnp.int32, 16)
            jf = jnp.full((16,), 0, jnp.int32) + j
            acc = jnp.zeros((16,), jnp.float32)
            for comp_i in range(3):
                cf = jnp.full((16,), comp_i, jnp.int32)
                av = plsc.load_gather(arows, [jf, rows, cf])
                bv = plsc.load_gather(brows, [jf, rows, cf])
                dv = av - bv
                acc = acc + dv * dv
            d2v[pl.ds(q * 16, 16)] = acc
            return carry2

        lax.fori_loop(0, _K * 8, compute, 0)
        pltpu.sync_copy(d2v, out_hbm.at[pl.ds(base, _CB)])
        return carry

    lax.fori_loop(0, nt, chunk_body, 0)


def _sc_d2(p4, src, dst):
    e = src.shape[0]
    nchunk = e // _CB
    mesh = plsc.VectorSubcoreMesh(
        core_axis_name="c", subcore_axis_name="s",
        num_cores=_NC, num_subcores=_NS)
    fn = pl.kernel(
        functools.partial(_sc_d2_body, nchunk=nchunk),
        out_type=jax.ShapeDtypeStruct((e,), jnp.float32),
        mesh=mesh,
        compiler_params=pltpu.CompilerParams(
            needs_layout_passes=False, use_tc_tiling_on_sc=False),
        scratch_types=[
            pltpu.VMEM((_CB,), jnp.int32),
            pltpu.VMEM((_CB,), jnp.int32),
            pltpu.VMEM((_K, _LANES, 4), jnp.float32),
            pltpu.VMEM((_K, _LANES, 4), jnp.float32),
            pltpu.VMEM((_CB,), jnp.float32),
            pltpu.SemaphoreType.DMA,
        ],
    )
    return fn(p4, src, dst)


def _split_body(ei_ref, src_ref, dst_ref):
    src_ref[...] = ei_ref[0, :]
    dst_ref[...] = ei_ref[1, :]


def _split_edges(edge_index, sb):
    e = edge_index.shape[1]
    out = jax.ShapeDtypeStruct((e,), jnp.int32)
    return pl.pallas_call(
        _split_body,
        grid=(e // sb,),
        in_specs=[pl.BlockSpec((2, sb), lambda i: (0, i))],
        out_specs=[pl.BlockSpec((sb,), lambda i: (i,)),
                   pl.BlockSpec((sb,), lambda i: (i,))],
        out_shape=[out, out],
    )(edge_index)


def _sin_poly(x):
    pi_hi = jnp.float32(3.1415927)
    pi_lo = jnp.float32(-8.742278e-8)
    k = jnp.floor(x * jnp.float32(1.0 / math.pi) + jnp.float32(0.5))
    r = x - k * pi_hi
    r = r - k * pi_lo
    sign = jnp.float32(1.0) - jnp.float32(2.0) * (
        k.astype(jnp.int32) & 1).astype(jnp.float32)
    r2 = r * r
    p = jnp.float32(2.7557314e-6)
    p = p * r2 + jnp.float32(-1.9841270e-4)
    p = p * r2 + jnp.float32(8.3333333e-3)
    p = p * r2 + jnp.float32(-1.6666667e-1)
    p = p * r2 + jnp.float32(1.0)
    return sign * r * p


def _rbf_body(d2_ref, fq_ref, out_ref, *, norm, nb, cb):
    d2 = d2_ref[...].reshape(1, cb)
    d = jnp.sqrt(d2)
    invd = lax.rsqrt(d2) * jnp.float32(norm)
    dn = (((1,), (0,)), ((), ()))
    arg = lax.dot_general(fq_ref[:, 0:1], d, dn,
                          preferred_element_type=jnp.float32)
    invdb = lax.dot_general(fq_ref[:, 1:2], invd, dn,
                            preferred_element_type=jnp.float32)
    out_ref[...] = _sin_poly(arg) * invdb


def _rbf_t(d2, fq, norm, cb):
    e = d2.shape[0]
    nb = fq.shape[0]
    return pl.pallas_call(
        functools.partial(_rbf_body, norm=norm, nb=nb, cb=cb),
        grid=(e // cb,),
        in_specs=[
            pl.BlockSpec((cb,), lambda i: (i,)),
            pl.BlockSpec((nb, 2), lambda i: (0, 0)),
        ],
        out_specs=pl.BlockSpec((nb, cb), lambda i: (0, i)),
        out_shape=jax.ShapeDtypeStruct((nb, e), jnp.float32),
    )(d2, fq)


def kernel(pos, edge_index, freqs):
    e = edge_index.shape[1]
    nb = freqs.shape[0]

    p4 = jnp.pad(pos, ((0, 0), (0, 1)))
    src, dst = _split_edges(edge_index, 128000)

    d2 = _sc_d2(p4, src, dst)

    fqc = (freqs * (1.0 / CUTOFF)).reshape(nb, 1)
    fq = jnp.concatenate([fqc, jnp.ones((nb, 1), jnp.float32)], axis=1)
    norm = math.sqrt(2.0 / CUTOFF)

    cb = 25600
    while e % cb:
        cb //= 2
    out_t = _rbf_t(d2, fq, norm, cb)
    return out_t.T


# speedup vs baseline: 39.0905x; 1.2746x over previous
import functools
import math

import jax
import jax.numpy as jnp
from jax import lax
from jax.experimental import pallas as pl
from jax.experimental.pallas import tpu as pltpu
from jax.experimental.pallas import tpu_sc as plsc

CUTOFF = 5.0

_NC = 2
_NS = 16
_NW = _NC * _NS

_K = 40
_LANES = 128
_CB = _K * _LANES


def _sc_d2_body(p4_hbm, src_hbm, dst_hbm, out_hbm,
                sidx, didx, arows, brows, d2v, sem, *, base_chunk, nchunk):
    wid = lax.axis_index("s") * _NC + lax.axis_index("c")
    nt = (nchunk - 1 - wid) // _NW + 1

    def chunk_body(t, carry):
        c = wid + t * _NW
        base = (base_chunk + c) * _CB
        pltpu.sync_copy(src_hbm.at[pl.ds(base, _CB)], sidx)
        pltpu.sync_copy(dst_hbm.at[pl.ds(base, _CB)], didx)

        def gather5(t5, carry2):
            cps = []
            for u in range(5):
                j = t5 * 5 + u
                cps.append(pltpu.make_async_copy(
                    p4_hbm.at[sidx.at[pl.ds(j * _LANES, _LANES)]],
                    arows.at[j], sem))
                cps.append(pltpu.make_async_copy(
                    p4_hbm.at[didx.at[pl.ds(j * _LANES, _LANES)]],
                    brows.at[j], sem))
            for cp in cps:
                cp.start()
            for cp in cps:
                cp.wait()
            return carry2

        lax.fori_loop(0, _K // 5, gather5, 0)

        def compute(q, carry2):
            j = q // 8
            l0 = (q % 8) * 16
            rows = l0 + lax.iota(jnp.int32, 16)
            jf = jnp.full((16,), 0, jnp.int32) + j
            acc = jnp.zeros((16,), jnp.float32)
            for comp_i in range(3):
                cf = jnp.full((16,), comp_i, jnp.int32)
                av = plsc.load_gather(arows, [jf, rows, cf])
                bv = plsc.load_gather(brows, [jf, rows, cf])
                dv = av - bv
                acc = acc + dv * dv
            d2v[pl.ds(q * 16, 16)] = acc
            return carry2

        lax.fori_loop(0, _K * 8, compute, 0)
        pltpu.sync_copy(d2v, out_hbm.at[pl.ds(c * _CB, _CB)])
        return carry

    lax.fori_loop(0, nt, chunk_body, 0)


def _sc_d2(p4, src, dst, base_chunk, nchunk):
    mesh = plsc.VectorSubcoreMesh(
        core_axis_name="c", subcore_axis_name="s",
        num_cores=_NC, num_subcores=_NS)
    fn = pl.kernel(
        functools.partial(_sc_d2_body, base_chunk=base_chunk, nchunk=nchunk),
        out_type=jax.ShapeDtypeStruct((nchunk * _CB,), jnp.float32),
        mesh=mesh,
        compiler_params=pltpu.CompilerParams(
            needs_layout_passes=False, use_tc_tiling_on_sc=False),
        scratch_types=[
            pltpu.VMEM((_CB,), jnp.int32),
            pltpu.VMEM((_CB,), jnp.int32),
            pltpu.VMEM((_K, _LANES, 4), jnp.float32),
            pltpu.VMEM((_K, _LANES, 4), jnp.float32),
            pltpu.VMEM((_CB,), jnp.float32),
            pltpu.SemaphoreType.DMA,
        ],
    )
    return fn(p4, src, dst)


def _split_body(ei_ref, src_ref, dst_ref):
    src_ref[...] = ei_ref[0, :]
    dst_ref[...] = ei_ref[1, :]


def _split_edges(edge_index, sb):
    e = edge_index.shape[1]
    out = jax.ShapeDtypeStruct((e,), jnp.int32)
    return pl.pallas_call(
        _split_body,
        grid=(e // sb,),
        in_specs=[pl.BlockSpec((2, sb), lambda i: (0, i))],
        out_specs=[pl.BlockSpec((sb,), lambda i: (i,)),
                   pl.BlockSpec((sb,), lambda i: (i,))],
        out_shape=[out, out],
    )(edge_index)


def _sin_poly(x):
    pi_hi = jnp.float32(3.1415927)
    pi_lo = jnp.float32(-8.742278e-8)
    k = jnp.floor(x * jnp.float32(1.0 / math.pi) + jnp.float32(0.5))
    r = x - k * pi_hi
    r = r - k * pi_lo
    sign = jnp.float32(1.0) - jnp.float32(2.0) * (
        k.astype(jnp.int32) & 1).astype(jnp.float32)
    r2 = r * r
    p = jnp.float32(2.7557314e-6)
    p = p * r2 + jnp.float32(-1.9841270e-4)
    p = p * r2 + jnp.float32(8.3333333e-3)
    p = p * r2 + jnp.float32(-1.6666667e-1)
    p = p * r2 + jnp.float32(1.0)
    return sign * r * p


def _rbf_body(d2_ref, fq_ref, out_ref, *, norm, nb, cb):
    d2 = d2_ref[...].reshape(1, cb)
    d = jnp.sqrt(d2)
    invd = lax.rsqrt(d2) * jnp.float32(norm)
    dn = (((1,), (0,)), ((), ()))
    arg = lax.dot_general(fq_ref[:, 0:1], d, dn,
                          preferred_element_type=jnp.float32)
    invdb = lax.dot_general(fq_ref[:, 1:2], invd, dn,
                            preferred_element_type=jnp.float32)
    out_ref[...] = _sin_poly(arg) * invdb


def _rbf_body_seg(d2_ref, fq_ref, prev_ref, out_ref, *, norm, nb, cb):
    del prev_ref
    _rbf_body(d2_ref, fq_ref, out_ref, norm=norm, nb=nb, cb=cb)


def _rbf_t_seg(d2_seg, fq, norm, cb, e_total, col0, prev):
    e_seg = d2_seg.shape[0]
    nb = fq.shape[0]
    blk0 = col0 // cb
    if prev is None:
        body = functools.partial(_rbf_body, norm=norm, nb=nb, cb=cb)
        in_specs = [
            pl.BlockSpec((cb,), lambda i: (i,)),
            pl.BlockSpec((nb, 2), lambda i: (0, 0)),
        ]
        args = (d2_seg, fq)
        aliases = {}
    else:
        body = functools.partial(_rbf_body_seg, norm=norm, nb=nb, cb=cb)
        in_specs = [
            pl.BlockSpec((cb,), lambda i: (i,)),
            pl.BlockSpec((nb, 2), lambda i: (0, 0)),
            pl.BlockSpec(memory_space=pl.ANY),
        ]
        args = (d2_seg, fq, prev)
        aliases = {2: 0}
    return pl.pallas_call(
        body,
        grid=(e_seg // cb,),
        in_specs=in_specs,
        out_specs=pl.BlockSpec((nb, cb), lambda i, b=blk0: (0, b + i)),
        out_shape=jax.ShapeDtypeStruct((nb, e_total), jnp.float32),
        input_output_aliases=aliases,
    )(*args)


def kernel(pos, edge_index, freqs):
    e = edge_index.shape[1]
    nb = freqs.shape[0]

    p4 = jnp.pad(pos, ((0, 0), (0, 1)))
    src, dst = _split_edges(edge_index, 128000)

    fqc = (freqs * (1.0 / CUTOFF)).reshape(nb, 1)
    fq = jnp.concatenate([fqc, jnp.ones((nb, 1), jnp.float32)], axis=1)
    norm = math.sqrt(2.0 / CUTOFF)

    nchunk = e // _CB
    nseg = 5
    while nchunk % nseg:
        nseg -= 1
    seg_chunks = nchunk // nseg
    e_seg = seg_chunks * _CB

    cb = 25600
    while e_seg % cb:
        cb //= 2

    d2_segs = [
        _sc_d2(p4, src, dst, s * seg_chunks, seg_chunks) for s in range(nseg)
    ]
    out_t = None
    for s in range(nseg):
        out_t = _rbf_t_seg(d2_segs[s], fq, norm, cb, e, s * e_seg, out_t)
    return out_t.T


# speedup vs baseline: 51.6177x; 1.3205x over previous
import functools
import math

import jax
import jax.numpy as jnp
from jax import lax
from jax.experimental import pallas as pl
from jax.experimental.pallas import tpu as pltpu
from jax.experimental.pallas import tpu_sc as plsc

CUTOFF = 5.0

_NC = 2
_NS = 16
_NW = _NC * _NS

_K = 20
_LANES = 128
_CB = _K * _LANES


def _sc_d2_body(p4_hbm, src_hbm, dst_hbm, out_hbm,
                sidx, didx, arows, brows, d2v, sem0, sem1,
                *, base_chunk, nchunk):
    wid = lax.axis_index("s") * _NC + lax.axis_index("c")
    nt = (nchunk - 1 - wid) // _NW + 1
    tmax = (nchunk + _NW - 1) // _NW
    sems = (sem0, sem1)

    def stage_idx(t):
        p = t & 1
        base = (base_chunk + wid + t * _NW) * _CB
        pltpu.sync_copy(src_hbm.at[pl.ds(base, _CB)], sidx.at[p])
        pltpu.sync_copy(dst_hbm.at[pl.ds(base, _CB)], didx.at[p])

    def _cps(t, j):
        p = t & 1
        return (
            pltpu.make_async_copy(
                p4_hbm.at[sidx.at[p, pl.ds(j * _LANES, _LANES)]],
                arows.at[p, j], sems[p]),
            pltpu.make_async_copy(
                p4_hbm.at[didx.at[p, pl.ds(j * _LANES, _LANES)]],
                brows.at[p, j], sems[p]),
        )

    def fire(t):
        def f(i, carry):
            for u in range(5):
                for cp in _cps(t, i * 5 + u):
                    cp.start()
            return carry
        lax.fori_loop(0, _K // 5, f, 0)

    def drain(t):
        def w(i, carry):
            for u in range(5):
                for cp in _cps(t, i * 5 + u):
                    cp.wait()
            return carry
        lax.fori_loop(0, _K // 5, w, 0)

    def compute_chunk(t):
        p = t & 1

        def compute(q, carry):
            j = q // 8
            l0 = (q % 8) * 16
            rows = l0 + lax.iota(jnp.int32, 16)
            jf = jnp.full((16,), 0, jnp.int32) + j
            pf = jnp.full((16,), p, jnp.int32)
            acc = jnp.zeros((16,), jnp.float32)
            for comp_i in range(3):
                cf = jnp.full((16,), comp_i, jnp.int32)
                av = plsc.load_gather(arows, [pf, jf, rows, cf])
                bv = plsc.load_gather(brows, [pf, jf, rows, cf])
                dv = av - bv
                acc = acc + dv * dv
            d2v[pl.ds(q * 16, 16)] = acc
            return carry

        lax.fori_loop(0, _K * 8, compute, 0)
        c_local = wid + t * _NW
        pltpu.sync_copy(d2v, out_hbm.at[pl.ds(c_local * _CB, _CB)])

    stage_idx(0)
    fire(0)
    for t in range(tmax):
        if t + 1 < tmax:
            @pl.when(t + 1 < nt)
            def _():
                stage_idx(t + 1)
                fire(t + 1)
        @pl.when(t < nt)
        def _():
            drain(t)
            compute_chunk(t)


def _sc_d2(p4, src, dst, base_chunk, nchunk):
    mesh = plsc.VectorSubcoreMesh(
        core_axis_name="c", subcore_axis_name="s",
        num_cores=_NC, num_subcores=_NS)
    fn = pl.kernel(
        functools.partial(_sc_d2_body, base_chunk=base_chunk, nchunk=nchunk),
        out_type=jax.ShapeDtypeStruct((nchunk * _CB,), jnp.float32),
        mesh=mesh,
        compiler_params=pltpu.CompilerParams(
            needs_layout_passes=False, use_tc_tiling_on_sc=False),
        scratch_types=[
            pltpu.VMEM((2, _CB), jnp.int32),
            pltpu.VMEM((2, _CB), jnp.int32),
            pltpu.VMEM((2, _K, _LANES, 4), jnp.float32),
            pltpu.VMEM((2, _K, _LANES, 4), jnp.float32),
            pltpu.VMEM((_CB,), jnp.float32),
            pltpu.SemaphoreType.DMA,
            pltpu.SemaphoreType.DMA,
        ],
    )
    return fn(p4, src, dst)


def _split_body(ei_ref, src_ref, dst_ref):
    src_ref[...] = ei_ref[0, :]
    dst_ref[...] = ei_ref[1, :]


def _split_edges(edge_index, sb):
    e = edge_index.shape[1]
    out = jax.ShapeDtypeStruct((e,), jnp.int32)
    return pl.pallas_call(
        _split_body,
        grid=(e // sb,),
        in_specs=[pl.BlockSpec((2, sb), lambda i: (0, i))],
        out_specs=[pl.BlockSpec((sb,), lambda i: (i,)),
                   pl.BlockSpec((sb,), lambda i: (i,))],
        out_shape=[out, out],
    )(edge_index)


def _sin_poly(x):
    pi_hi = jnp.float32(3.1415927)
    pi_lo = jnp.float32(-8.742278e-8)
    k = jnp.floor(x * jnp.float32(1.0 / math.pi) + jnp.float32(0.5))
    r = x - k * pi_hi
    r = r - k * pi_lo
    sign = jnp.float32(1.0) - jnp.float32(2.0) * (
        k.astype(jnp.int32) & 1).astype(jnp.float32)
    r2 = r * r
    p = jnp.float32(2.7557314e-6)
    p = p * r2 + jnp.float32(-1.9841270e-4)
    p = p * r2 + jnp.float32(8.3333333e-3)
    p = p * r2 + jnp.float32(-1.6666667e-1)
    p = p * r2 + jnp.float32(1.0)
    return sign * r * p


def _rbf_body(d2_ref, fq_ref, out_ref, *, norm, nb, cb):
    d2 = d2_ref[...].reshape(1, cb)
    d = jnp.sqrt(d2)
    invd = lax.rsqrt(d2) * jnp.float32(norm)
    dn = (((1,), (0,)), ((), ()))
    arg = lax.dot_general(fq_ref[:, 0:1], d, dn,
                          preferred_element_type=jnp.float32)
    invdb = lax.dot_general(fq_ref[:, 1:2], invd, dn,
                            preferred_element_type=jnp.float32)
    out_ref[...] = _sin_poly(arg) * invdb


def _rbf_body_seg(d2_ref, fq_ref, prev_ref, out_ref, *, norm, nb, cb):
    del prev_ref
    _rbf_body(d2_ref, fq_ref, out_ref, norm=norm, nb=nb, cb=cb)


def _rbf_t_seg(d2_seg, fq, norm, cb, e_total, col0, prev):
    e_seg = d2_seg.shape[0]
    nb = fq.shape[0]
    blk0 = col0 // cb
    if prev is None:
        body = functools.partial(_rbf_body, norm=norm, nb=nb, cb=cb)
        in_specs = [
            pl.BlockSpec((cb,), lambda i: (i,)),
            pl.BlockSpec((nb, 2), lambda i: (0, 0)),
        ]
        args = (d2_seg, fq)
        aliases = {}
    else:
        body = functools.partial(_rbf_body_seg, norm=norm, nb=nb, cb=cb)
        in_specs = [
            pl.BlockSpec((cb,), lambda i: (i,)),
            pl.BlockSpec((nb, 2), lambda i: (0, 0)),
            pl.BlockSpec(memory_space=pl.ANY),
        ]
        args = (d2_seg, fq, prev)
        aliases = {2: 0}
    return pl.pallas_call(
        body,
        grid=(e_seg // cb,),
        in_specs=in_specs,
        out_specs=pl.BlockSpec((nb, cb), lambda i, b=blk0: (0, b + i)),
        out_shape=jax.ShapeDtypeStruct((nb, e_total), jnp.float32),
        input_output_aliases=aliases,
    )(*args)


def kernel(pos, edge_index, freqs):
    e = edge_index.shape[1]
    nb = freqs.shape[0]

    p4 = jnp.pad(pos, ((0, 0), (0, 1)))
    src, dst = _split_edges(edge_index, 128000)

    fqc = (freqs * (1.0 / CUTOFF)).reshape(nb, 1)
    fq = jnp.concatenate([fqc, jnp.ones((nb, 1), jnp.float32)], axis=1)
    norm = math.sqrt(2.0 / CUTOFF)

    nchunk = e // _CB
    nseg = 5
    while nchunk % nseg:
        nseg -= 1
    seg_chunks = nchunk // nseg
    e_seg = seg_chunks * _CB

    cb = 25600
    while e_seg % cb:
        cb //= 2

    d2_segs = [
        _sc_d2(p4, src, dst, s * seg_chunks, seg_chunks) for s in range(nseg)
    ]
    out_t = None
    for s in range(nseg):
        out_t = _rbf_t_seg(d2_segs[s], fq, norm, cb, e, s * e_seg, out_t)
    return out_t.T


# speedup vs baseline: 52.9912x; 1.0266x over previous
import functools
import math

import jax
import jax.numpy as jnp
from jax import lax
from jax.experimental import pallas as pl
from jax.experimental.pallas import tpu as pltpu
from jax.experimental.pallas import tpu_sc as plsc

CUTOFF = 5.0

_NC = 2
_NS = 16
_NW = _NC * _NS

_K = 20
_LANES = 128
_CB = _K * _LANES


def _sc_d2_body(p4_hbm, src_hbm, dst_hbm, out_hbm,
                sidx, didx, arows, brows, d2v, sem0, sem1,
                *, base_chunk, nchunk):
    wid = lax.axis_index("s") * _NC + lax.axis_index("c")
    nt = (nchunk - 1 - wid) // _NW + 1
    tmax = (nchunk + _NW - 1) // _NW
    sems = (sem0, sem1)

    def stage_idx(t):
        p = t & 1
        base = (base_chunk + wid + t * _NW) * _CB
        pltpu.sync_copy(src_hbm.at[pl.ds(base, _CB)], sidx.at[p])
        pltpu.sync_copy(dst_hbm.at[pl.ds(base, _CB)], didx.at[p])

    def _cps(t, j):
        p = t & 1
        return (
            pltpu.make_async_copy(
                p4_hbm.at[sidx.at[p, pl.ds(j * _LANES, _LANES)]],
                arows.at[p, j], sems[p]),
            pltpu.make_async_copy(
                p4_hbm.at[didx.at[p, pl.ds(j * _LANES, _LANES)]],
                brows.at[p, j], sems[p]),
        )

    def fire(t):
        def f(i, carry):
            for u in range(5):
                for cp in _cps(t, i * 5 + u):
                    cp.start()
            return carry
        lax.fori_loop(0, _K // 5, f, 0)

    def drain(t):
        def w(i, carry):
            for u in range(5):
                for cp in _cps(t, i * 5 + u):
                    cp.wait()
            return carry
        lax.fori_loop(0, _K // 5, w, 0)

    def compute_chunk(t):
        p = t & 1

        def compute(q, carry):
            j = q // 8
            l0 = (q % 8) * 16
            rows = l0 + lax.iota(jnp.int32, 16)
            jf = jnp.full((16,), 0, jnp.int32) + j
            pf = jnp.full((16,), p, jnp.int32)
            acc = jnp.zeros((16,), jnp.float32)
            for comp_i in range(3):
                cf = jnp.full((16,), comp_i, jnp.int32)
                av = plsc.load_gather(arows, [pf, jf, rows, cf])
                bv = plsc.load_gather(brows, [pf, jf, rows, cf])
                dv = av - bv
                acc = acc + dv * dv
            d2v[pl.ds(q * 16, 16)] = acc
            return carry

        lax.fori_loop(0, _K * 8, compute, 0)
        c_local = wid + t * _NW
        pltpu.sync_copy(d2v, out_hbm.at[pl.ds(c_local * _CB, _CB)])

    stage_idx(0)
    fire(0)
    for t in range(tmax):
        if t + 1 < tmax:
            @pl.when(t + 1 < nt)
            def _():
                stage_idx(t + 1)
                fire(t + 1)
        @pl.when(t < nt)
        def _():
            drain(t)
            compute_chunk(t)


def _sc_d2(p4, src, dst, base_chunk, nchunk):
    mesh = plsc.VectorSubcoreMesh(
        core_axis_name="c", subcore_axis_name="s",
        num_cores=_NC, num_subcores=_NS)
    fn = pl.kernel(
        functools.partial(_sc_d2_body, base_chunk=base_chunk, nchunk=nchunk),
        out_type=jax.ShapeDtypeStruct((nchunk * _CB,), jnp.float32),
        mesh=mesh,
        compiler_params=pltpu.CompilerParams(
            needs_layout_passes=False, use_tc_tiling_on_sc=False),
        scratch_types=[
            pltpu.VMEM((2, _CB), jnp.int32),
            pltpu.VMEM((2, _CB), jnp.int32),
            pltpu.VMEM((2, _K, _LANES, 4), jnp.float32),
            pltpu.VMEM((2, _K, _LANES, 4), jnp.float32),
            pltpu.VMEM((_CB,), jnp.float32),
            pltpu.SemaphoreType.DMA,
            pltpu.SemaphoreType.DMA,
        ],
    )
    return fn(p4, src, dst)


def _split_body(ei_ref, src_ref, dst_ref):
    src_ref[...] = ei_ref[0, :]
    dst_ref[...] = ei_ref[1, :]


def _split_edges(edge_index, sb, e0, es):
    b0 = e0 // sb
    out = jax.ShapeDtypeStruct((es,), jnp.int32)
    return pl.pallas_call(
        _split_body,
        grid=(es // sb,),
        in_specs=[pl.BlockSpec((2, sb), lambda i, b=b0: (0, b + i))],
        out_specs=[pl.BlockSpec((sb,), lambda i: (i,)),
                   pl.BlockSpec((sb,), lambda i: (i,))],
        out_shape=[out, out],
    )(edge_index)


def _sin_poly(x):
    pi_hi = jnp.float32(3.1415927)
    pi_lo = jnp.float32(-8.742278e-8)
    k = jnp.floor(x * jnp.float32(1.0 / math.pi) + jnp.float32(0.5))
    r = x - k * pi_hi
    r = r - k * pi_lo
    sign = jnp.float32(1.0) - jnp.float32(2.0) * (
        k.astype(jnp.int32) & 1).astype(jnp.float32)
    r2 = r * r
    p = jnp.float32(2.7557314e-6)
    p = p * r2 + jnp.float32(-1.9841270e-4)
    p = p * r2 + jnp.float32(8.3333333e-3)
    p = p * r2 + jnp.float32(-1.6666667e-1)
    p = p * r2 + jnp.float32(1.0)
    return sign * r * p


def _rbf_body(d2_ref, fq_ref, out_ref, *, norm, nb, cb):
    d2 = d2_ref[...].reshape(1, cb)
    d = jnp.sqrt(d2)
    invd = lax.rsqrt(d2) * jnp.float32(norm)
    dn = (((1,), (0,)), ((), ()))
    arg = lax.dot_general(fq_ref[:, 0:1], d, dn,
                          preferred_element_type=jnp.float32)
    invdb = lax.dot_general(fq_ref[:, 1:2], invd, dn,
                            preferred_element_type=jnp.float32)
    out_ref[...] = _sin_poly(arg) * invdb


def _rbf_body_seg(d2_ref, fq_ref, prev_ref, out_ref, *, norm, nb, cb):
    del prev_ref
    _rbf_body(d2_ref, fq_ref, out_ref, norm=norm, nb=nb, cb=cb)


def _rbf_t_seg(d2_seg, fq, norm, cb, e_total, col0, prev):
    e_seg = d2_seg.shape[0]
    nb = fq.shape[0]
    blk0 = col0 // cb
    if prev is None:
        body = functools.partial(_rbf_body, norm=norm, nb=nb, cb=cb)
        in_specs = [
            pl.BlockSpec((cb,), lambda i: (i,)),
            pl.BlockSpec((nb, 2), lambda i: (0, 0)),
        ]
        args = (d2_seg, fq)
        aliases = {}
    else:
        body = functools.partial(_rbf_body_seg, norm=norm, nb=nb, cb=cb)
        in_specs = [
            pl.BlockSpec((cb,), lambda i: (i,)),
            pl.BlockSpec((nb, 2), lambda i: (0, 0)),
            pl.BlockSpec(memory_space=pl.ANY),
        ]
        args = (d2_seg, fq, prev)
        aliases = {2: 0}
    return pl.pallas_call(
        body,
        grid=(e_seg // cb,),
        in_specs=in_specs,
        out_specs=pl.BlockSpec((nb, cb), lambda i, b=blk0: (0, b + i)),
        out_shape=jax.ShapeDtypeStruct((nb, e_total), jnp.float32),
        input_output_aliases=aliases,
    )(*args)


def kernel(pos, edge_index, freqs):
    e = edge_index.shape[1]
    nb = freqs.shape[0]

    p4 = jnp.pad(pos, ((0, 0), (0, 1)))

    fqc = (freqs * (1.0 / CUTOFF)).reshape(nb, 1)
    fq = jnp.concatenate([fqc, jnp.ones((nb, 1), jnp.float32)], axis=1)
    norm = math.sqrt(2.0 / CUTOFF)

    nchunk = e // _CB
    unit = 50
    if nchunk % unit == 0 and nchunk // unit >= 5:
        units = nchunk // unit
        rem = units - 2
        q, r = divmod(rem, 4)
        sizes = [(q + (1 if i < r else 0)) * unit for i in range(4)]
        sizes.append(2 * unit)
    else:
        sizes = [nchunk]
    sb = 128000
    cb = 25600

    d2_segs = []
    base = 0
    bases = []
    for sc in sizes:
        es = sc * _CB
        srcs, dsts = _split_edges(edge_index, sb, base * _CB, es)
        d2_segs.append(_sc_d2(p4, srcs, dsts, 0, sc))
        bases.append(base)
        base += sc
    out_t = None
    for s in range(len(sizes)):
        out_t = _rbf_t_seg(d2_segs[s], fq, norm, cb, e, bases[s] * _CB, out_t)
    return out_t.T
